# Initial kernel scaffold; baseline (speedup 1.0000x reference)
import jax
import jax.numpy as jnp
from jax.experimental import pallas as pl


def kernel(x, edge_index, W1, as1, ad1, b1, W2, as2, ad2, b2, W3, as3, ad3, b3):
    raise NotImplementedError("write your pallas kernel here")


# speedup vs baseline: 1.5821x; 1.5821x over previous
import functools

import jax
import jax.numpy as jnp
from jax.experimental import pallas as pl
from jax.experimental.pallas import tpu as pltpu

N_NODES = 10000
ROW_BLK = 1000


def _prep_body(x_ref, w_ref, acat_ref, h_ref, aa_ref):
    h = jnp.dot(x_ref[...], w_ref[...], preferred_element_type=jnp.float32)
    h_ref[...] = h
    aa_ref[...] = jnp.dot(h, acat_ref[...], preferred_element_type=jnp.float32)


def _prep(x, W, acat):
    n, _ = x.shape
    c = W.shape[1]
    grid = (n // ROW_BLK,)
    h, aa = pl.pallas_call(
        _prep_body,
        grid=grid,
        in_specs=[
            pl.BlockSpec((ROW_BLK, x.shape[1]), lambda i: (i, 0)),
            pl.BlockSpec(W.shape, lambda i: (0, 0)),
            pl.BlockSpec(acat.shape, lambda i: (0, 0)),
        ],
        out_specs=[
            pl.BlockSpec((ROW_BLK, c), lambda i: (i, 0)),
            pl.BlockSpec((ROW_BLK, 128), lambda i: (i, 0)),
        ],
        out_shape=[
            jax.ShapeDtypeStruct((n, c), jnp.float32),
            jax.ShapeDtypeStruct((n, 128), jnp.float32),
        ],
    )(x, W, acat)
    return h, aa[:, 0], aa[:, 1]


def _edge_phase(h, asn, adn, src, dst, n_nodes):
    e = asn[src] + adn[dst]
    e = jnp.maximum(e, 0.2 * e)
    w = jnp.exp(e)
    den = jax.ops.segment_sum(w, dst, num_segments=n_nodes)
    msg = h[src] * w[:, None]
    num = jax.ops.segment_sum(msg, dst, num_segments=n_nodes)
    return num, den


def _acat(a_src, a_dst, c):
    a = jnp.zeros((c, 128), jnp.float32)
    a = a.at[:, 0].set(a_src.reshape(-1))
    a = a.at[:, 1].set(a_dst.reshape(-1))
    return a


def kernel(x, edge_index, W1, as1, ad1, b1, W2, as2, ad2, b2, W3, as3, ad3, b3):
    src = edge_index[0].astype(jnp.int32)
    dst = edge_index[1].astype(jnp.int32)
    loop = jnp.arange(N_NODES, dtype=jnp.int32)
    src = jnp.concatenate([src, loop])
    dst = jnp.concatenate([dst, loop])

    h, asn, adn = _prep(x, W1, _acat(as1, ad1, 128))
    num, den = _edge_phase(h, asn, adn, src, dst, N_NODES)
    h1 = jax.nn.relu(num / (den[:, None] + 1e-16) + b1)

    h, asn, adn = _prep(h1, W2, _acat(as2, ad2, 128))
    num, den = _edge_phase(h, asn, adn, src, dst, N_NODES)
    h2 = jax.nn.relu(num / (den[:, None] + 1e-16) + b2)

    h, asn, adn = _prep(h2, W3, _acat(as3, ad3, 64))
    num, den = _edge_phase(h, asn, adn, src, dst, N_NODES)
    return num / (den[:, None] + 1e-16) + b3
